# baseline (speedup 1.0000x reference)
import functools

import jax
import jax.numpy as jnp
from jax import lax
from jax.experimental import pallas as pl
from jax.experimental.pallas import tpu as pltpu
from jax.experimental.pallas import tpu_sc as plsc

NUM_CORES = 2
NUM_SUBCORES = 16
LANES = 16
NUM_WORKERS = NUM_CORES * NUM_SUBCORES

BATCH = 16384
FACTORS = 32
B_PER_W = BATCH // NUM_WORKERS


def _make_sc_kernel():
  mesh = plsc.VectorSubcoreMesh(core_axis_name="c", subcore_axis_name="s")

  @functools.partial(
      pl.kernel,
      out_type=jax.ShapeDtypeStruct((BATCH,), jnp.float32),
      mesh=mesh,
      compiler_params=pltpu.CompilerParams(use_tc_tiling_on_sc=False),
      scratch_types=[
          pltpu.VMEM((B_PER_W,), jnp.int32),
          pltpu.VMEM((B_PER_W,), jnp.int32),
          pltpu.VMEM((FACTORS, B_PER_W), jnp.float32),
          pltpu.VMEM((FACTORS, B_PER_W), jnp.float32),
          pltpu.VMEM((B_PER_W,), jnp.float32),
          pltpu.SemaphoreType.DMA,
          pltpu.SemaphoreType.DMA,
      ],
  )
  def dot_kernel(inv_idx_hbm, td_idx_hbm, inv_t_hbm, td_t_hbm, out_hbm,
                 idx_a, idx_b, fa, fb, out_v, sem_a, sem_b):
    wid = lax.axis_index("s") * NUM_CORES + lax.axis_index("c")
    base = wid * B_PER_W

    pltpu.sync_copy(inv_idx_hbm.at[pl.ds(base, B_PER_W)], idx_a)
    pltpu.sync_copy(td_idx_hbm.at[pl.ds(base, B_PER_W)], idx_b)

    cps = []
    for d in range(FACTORS):
      cps.append(pltpu.async_copy(inv_t_hbm.at[d].at[idx_a], fa.at[d], sem_a))
      cps.append(pltpu.async_copy(td_t_hbm.at[d].at[idx_b], fb.at[d], sem_b))
    for cp in cps:
      cp.wait()

    def group_body(g):
      sl = pl.ds(g * LANES, LANES)
      acc = jnp.zeros((LANES,), jnp.float32)
      for d in range(FACTORS):
        acc = acc + fa[d, sl] * fb[d, sl]
      out_v[sl] = acc

    pl.loop(0, B_PER_W // LANES)(group_body)

    pltpu.sync_copy(out_v, out_hbm.at[pl.ds(base, B_PER_W)])

  return dot_kernel


_sc_dot = _make_sc_kernel()


@jax.jit
def kernel(investor, ticker, date, ticker_date, investor_factors,
           ticker_date_factors):
  del ticker, date
  inv_idx = investor.astype(jnp.int32)
  td_idx = ticker_date.astype(jnp.int32)
  return _sc_dot(inv_idx, td_idx, investor_factors.T, ticker_date_factors.T)
